# Initial kernel scaffold; baseline (speedup 1.0000x reference)
import jax
import jax.numpy as jnp
from jax.experimental import pallas as pl


def kernel(x, edge_index, edge_weight, W1, b1, W2, b2):
    raise NotImplementedError("write your pallas kernel here")


# speedup vs baseline: 34.2052x; 34.2052x over previous
import functools

import jax
import jax.numpy as jnp
from jax import lax
from jax.experimental import pallas as pl
from jax.experimental.pallas import tpu as pltpu
from jax.experimental.pallas import tpu_sc as plsc

NC = 2
NS = 16
NW = NC * NS
L = 16
CHUNK = 128

N_PAD = 10240
ROWS_PER_TILE = N_PAD // NS


def _sc_mesh():
    return plsc.VectorSubcoreMesh(
        core_axis_name="c", subcore_axis_name="s", num_cores=NC,
        num_subcores=NS)


def _make_deg_kernel(cpw):
    @functools.partial(
        pl.kernel,
        out_type=jax.ShapeDtypeStruct((NC, N_PAD), jnp.float32),
        mesh=_sc_mesh(),
        scratch_types=[
            pltpu.VMEM((cpw, CHUNK), jnp.int32),
            pltpu.VMEM((cpw, CHUNK), jnp.float32),
            pltpu.VMEM((ROWS_PER_TILE,), jnp.float32),
            pltpu.VMEM_SHARED((N_PAD,), jnp.float32),
        ],
    )
    def deg_kernel(dst_hbm, ew_hbm, out_hbm, idx_v, ew_v, z_v, acc_sh):
        c = lax.axis_index("c")
        s = lax.axis_index("s")
        w = c * NS + s

        def zfill(i, _):
            z_v[pl.ds(i * L, L)] = jnp.zeros((L,), jnp.float32)
            return 0
        lax.fori_loop(0, ROWS_PER_TILE // L, zfill, 0)
        pltpu.sync_copy(z_v, acc_sh.at[pl.ds(s * ROWS_PER_TILE,
                                             ROWS_PER_TILE)])
        plsc.subcore_barrier()

        pltpu.sync_copy(dst_hbm.at[w], idx_v)
        pltpu.sync_copy(ew_hbm.at[w], ew_v)

        def body(k, _):
            pltpu.sync_copy(ew_v.at[k], acc_sh.at[idx_v.at[k]], add=True)
            return 0
        lax.fori_loop(0, cpw, body, 0)

        plsc.subcore_barrier()
        pltpu.sync_copy(acc_sh.at[pl.ds(s * ROWS_PER_TILE, ROWS_PER_TILE)],
                        out_hbm.at[c, pl.ds(s * ROWS_PER_TILE,
                                            ROWS_PER_TILE)])

    return deg_kernel


def _make_agg_kernel(cpw, width, n_rows):
    @functools.partial(
        pl.kernel,
        out_type=jax.ShapeDtypeStruct((NC, N_PAD, width), jnp.float32),
        mesh=_sc_mesh(),
        scratch_types=[
            pltpu.VMEM((cpw, CHUNK), jnp.int32),
            pltpu.VMEM((cpw, CHUNK), jnp.int32),
            pltpu.VMEM((cpw, CHUNK), jnp.float32),
            pltpu.VMEM((CHUNK, width), jnp.float32),
            pltpu.VMEM_SHARED((N_PAD, width), jnp.float32),
            pltpu.SemaphoreType.DMA,
        ],
        compiler_params=pltpu.CompilerParams(use_tc_tiling_on_sc=False),
    )
    def agg_kernel(table_hbm, src_hbm, dst_hbm, ew_hbm, out_hbm,
                   src_v, dst_v, ew_v, rows_v, acc_sh, sem):
        c = lax.axis_index("c")
        s = lax.axis_index("s")
        w = c * NS + s

        for g in range(CHUNK):
            for j in range(width // L):
                rows_v[g, pl.ds(j * L, L)] = jnp.zeros((L,), jnp.float32)
        for i in range(ROWS_PER_TILE // CHUNK):
            pltpu.sync_copy(
                rows_v,
                acc_sh.at[pl.ds(s * ROWS_PER_TILE + i * CHUNK, CHUNK)])
        plsc.subcore_barrier()

        pltpu.sync_copy(src_hbm.at[w], src_v)
        pltpu.sync_copy(dst_hbm.at[w], dst_v)
        pltpu.sync_copy(ew_hbm.at[w], ew_v)

        def body(k, _):
            pltpu.async_copy(table_hbm.at[src_v.at[k]], rows_v, sem).wait()
            for g in range(CHUNK // L):
                ewv = ew_v[k, pl.ds(g * L, L)]
                for i in range(L):
                    wv = ewv[i]
                    row = g * L + i
                    for j in range(width // L):
                        sl = rows_v[row, pl.ds(j * L, L)]
                        rows_v[row, pl.ds(j * L, L)] = sl * wv
            pltpu.sync_copy(rows_v, acc_sh.at[dst_v.at[k]], add=True)
            return 0
        lax.fori_loop(0, cpw, body, 0)

        plsc.subcore_barrier()
        pltpu.sync_copy(
            acc_sh.at[pl.ds(s * ROWS_PER_TILE, ROWS_PER_TILE)],
            out_hbm.at[c, pl.ds(s * ROWS_PER_TILE, ROWS_PER_TILE)])

    return agg_kernel


def _tc_prep(deg_p, x, W1, n):
    h = W1.shape[1]

    def body(degp_ref, x_ref, w1_ref, xws_ref):
        dp = degp_ref[...]
        deg = dp[0] + dp[1] + 1.0
        dis = lax.rsqrt(deg)[:n]
        xw = jnp.dot(x_ref[...], w1_ref[...],
                     preferred_element_type=jnp.float32)
        xws_ref[...] = xw * dis[:, None]

    return pl.pallas_call(
        body,
        out_shape=jax.ShapeDtypeStruct((n, h), jnp.float32),
    )(deg_p, x, W1)


def _tc_mid(s1_p, xws, deg_p, b1, W2p, n):
    wpad = W2p.shape[1]

    def body(s1p_ref, xws_ref, degp_ref, b1_ref, w2_ref, hws_ref):
        dp = degp_ref[...]
        deg = dp[0] + dp[1] + 1.0
        dis = lax.rsqrt(deg)[:n]
        s1 = s1p_ref[0, :n] + s1p_ref[1, :n]
        out1 = (s1 + xws_ref[...]) * dis[:, None] + b1_ref[...]
        h = jnp.maximum(out1, 0.0)
        hw = jnp.dot(h, w2_ref[...], preferred_element_type=jnp.float32)
        hws_ref[...] = hw * dis[:, None]

    return pl.pallas_call(
        body,
        out_shape=jax.ShapeDtypeStruct((n, wpad), jnp.float32),
    )(s1_p, xws, deg_p, b1, W2p)


def _tc_final(s2_p, hws, deg_p, b2, n, out_dim):
    def body(s2p_ref, hws_ref, degp_ref, b2_ref, out_ref):
        dp = degp_ref[...]
        deg = dp[0] + dp[1] + 1.0
        dis = lax.rsqrt(deg)[:n]
        s2 = s2p_ref[0, :n, :out_dim] + s2p_ref[1, :n, :out_dim]
        out_ref[...] = ((s2 + hws_ref[:, :out_dim]) * dis[:, None]
                        + b2_ref[...])

    return pl.pallas_call(
        body,
        out_shape=jax.ShapeDtypeStruct((n, out_dim), jnp.float32),
    )(s2_p, hws, deg_p, b2)


def kernel(x, edge_index, edge_weight, W1, b1, W2, b2):
    n, d = x.shape
    e = edge_index.shape[1]
    h = W1.shape[1]
    out_dim = W2.shape[1]
    w2pad = 16

    src = edge_index[0].astype(jnp.int32)
    dst = edge_index[1].astype(jnp.int32)
    ew = edge_weight.astype(jnp.float32)

    per_block = NW * CHUNK
    cpw = -(-e // per_block)
    e_pad = cpw * per_block
    npad = e_pad - e
    pad_ar = jnp.arange(npad, dtype=jnp.int32)
    src_p = jnp.concatenate([src, pad_ar % n]).reshape(NW, cpw, CHUNK)
    dst_p = jnp.concatenate([dst, n + pad_ar % (N_PAD - n)]
                            ).reshape(NW, cpw, CHUNK)
    ew_p = jnp.concatenate([ew, jnp.zeros((npad,), jnp.float32)]
                           ).reshape(NW, cpw, CHUNK)

    deg_p = _make_deg_kernel(cpw)(dst_p, ew_p)
    xws = _tc_prep(deg_p, x, W1, n)
    s1_p = _make_agg_kernel(cpw, h, n)(xws, src_p, dst_p, ew_p)
    W2p = jnp.concatenate(
        [W2, jnp.zeros((h, w2pad - out_dim), jnp.float32)], axis=1)
    hws = _tc_mid(s1_p, xws, deg_p, b1.reshape(1, h), W2p, n)
    s2_p = _make_agg_kernel(cpw, w2pad, n)(hws, src_p, dst_p, ew_p)
    return _tc_final(s2_p, hws, deg_p, b2.reshape(1, out_dim), n, out_dim)


# speedup vs baseline: 60.1502x; 1.7585x over previous
import functools

import jax
import jax.numpy as jnp
from jax import lax
from jax.experimental import pallas as pl
from jax.experimental.pallas import tpu as pltpu
from jax.experimental.pallas import tpu_sc as plsc

NC = 2
NS = 16
NW = NC * NS
L = 16
CHUNK = 128

N_PAD = 10240
ROWS_PER_TILE = N_PAD // NS


def _sc_mesh():
    return plsc.VectorSubcoreMesh(
        core_axis_name="c", subcore_axis_name="s", num_cores=NC,
        num_subcores=NS)


def _make_deg_kernel(cpw):
    @functools.partial(
        pl.kernel,
        out_type=jax.ShapeDtypeStruct((NC, N_PAD), jnp.float32),
        mesh=_sc_mesh(),
        scratch_types=[
            pltpu.VMEM((cpw, CHUNK), jnp.int32),
            pltpu.VMEM((cpw, CHUNK), jnp.float32),
            pltpu.VMEM((ROWS_PER_TILE,), jnp.float32),
            pltpu.VMEM_SHARED((N_PAD,), jnp.float32),
        ],
    )
    def deg_kernel(dst_hbm, ew_hbm, out_hbm, idx_v, ew_v, z_v, acc_sh):
        c = lax.axis_index("c")
        s = lax.axis_index("s")
        w = c * NS + s

        def zfill(i, _):
            z_v[pl.ds(i * L, L)] = jnp.zeros((L,), jnp.float32)
            return 0
        lax.fori_loop(0, ROWS_PER_TILE // L, zfill, 0)
        pltpu.sync_copy(z_v, acc_sh.at[pl.ds(s * ROWS_PER_TILE,
                                             ROWS_PER_TILE)])
        plsc.subcore_barrier()

        pltpu.sync_copy(dst_hbm.at[w], idx_v)
        pltpu.sync_copy(ew_hbm.at[w], ew_v)

        def body(k, _):
            pltpu.sync_copy(ew_v.at[k], acc_sh.at[idx_v.at[k]], add=True)
            return 0
        lax.fori_loop(0, cpw, body, 0)

        plsc.subcore_barrier()
        pltpu.sync_copy(acc_sh.at[pl.ds(s * ROWS_PER_TILE, ROWS_PER_TILE)],
                        out_hbm.at[c, pl.ds(s * ROWS_PER_TILE,
                                            ROWS_PER_TILE)])

    return deg_kernel


NBUF = 4


def _make_agg_kernel(cpw, width, n_rows):
    assert cpw % NBUF == 0

    @functools.partial(
        pl.kernel,
        out_type=jax.ShapeDtypeStruct((NC, N_PAD, width), jnp.float32),
        mesh=_sc_mesh(),
        scratch_types=[
            pltpu.VMEM((cpw, CHUNK), jnp.int32),
            pltpu.VMEM((cpw, CHUNK), jnp.int32),
            pltpu.VMEM((cpw, CHUNK), jnp.float32),
            [pltpu.VMEM((CHUNK, width), jnp.float32) for _ in range(NBUF)],
            [pltpu.VMEM((CHUNK, width), jnp.float32) for _ in range(NBUF)],
            pltpu.VMEM_SHARED((N_PAD, width), jnp.float32),
            [pltpu.SemaphoreType.DMA for _ in range(NBUF)],
            [pltpu.SemaphoreType.DMA for _ in range(NBUF)],
        ],
        compiler_params=pltpu.CompilerParams(use_tc_tiling_on_sc=False),
    )
    def agg_kernel(table_hbm, src_hbm, dst_hbm, ew_hbm, out_hbm,
                   src_v, dst_v, ew_v, gbufs, sbufs, acc_sh, gsems, ssems):
        c = lax.axis_index("c")
        s = lax.axis_index("s")
        w = c * NS + s
        rounds = cpw // NBUF

        for g in range(CHUNK):
            for j in range(width // L):
                gbufs[0][g, pl.ds(j * L, L)] = jnp.zeros((L,), jnp.float32)
        for i in range(ROWS_PER_TILE // CHUNK):
            pltpu.sync_copy(
                gbufs[0],
                acc_sh.at[pl.ds(s * ROWS_PER_TILE + i * CHUNK, CHUNK)])
        plsc.subcore_barrier()

        pltpu.sync_copy(src_hbm.at[w], src_v)
        pltpu.sync_copy(dst_hbm.at[w], dst_v)
        pltpu.sync_copy(ew_hbm.at[w], ew_v)

        def gather(k, j):
            pltpu.make_async_copy(
                table_hbm.at[src_v.at[k]], gbufs[j], gsems[j]).start()

        def scale(k, j):
            for g in range(CHUNK // L):
                ewv = ew_v[k, pl.ds(g * L, L)]
                for i in range(L):
                    wv = ewv[i]
                    row = g * L + i
                    for jj in range(width // L):
                        sl = gbufs[j][row, pl.ds(jj * L, L)]
                        sbufs[j][row, pl.ds(jj * L, L)] = sl * wv

        for j in range(NBUF):
            gather(j, j)

        def body(i, _):
            for j in range(NBUF):
                k = i * NBUF + j
                pltpu.make_async_copy(
                    table_hbm.at[src_v.at[k]], gbufs[j], gsems[j]).wait()

                @pl.when(i > 0)
                def _():
                    pltpu.make_async_copy(
                        sbufs[j], acc_sh.at[dst_v.at[k - NBUF]],
                        ssems[j]).wait()

                scale(k, j)

                @pl.when(i + 1 < rounds)
                def _():
                    gather(k + NBUF, j)

                pltpu.make_async_copy(
                    sbufs[j], acc_sh.at[dst_v.at[k]], ssems[j]
                ).start(add=True)
            return 0
        lax.fori_loop(0, rounds, body, 0)

        for j in range(NBUF):
            k = cpw - NBUF + j
            pltpu.make_async_copy(
                sbufs[j], acc_sh.at[dst_v.at[k]], ssems[j]).wait()

        plsc.subcore_barrier()
        pltpu.sync_copy(
            acc_sh.at[pl.ds(s * ROWS_PER_TILE, ROWS_PER_TILE)],
            out_hbm.at[c, pl.ds(s * ROWS_PER_TILE, ROWS_PER_TILE)])

    return agg_kernel


def _tc_prep(deg_p, x, W1, n):
    h = W1.shape[1]

    def body(degp_ref, x_ref, w1_ref, xws_ref):
        dp = degp_ref[...]
        deg = dp[0] + dp[1] + 1.0
        dis = lax.rsqrt(deg)[:n]
        xw = jnp.dot(x_ref[...], w1_ref[...],
                     preferred_element_type=jnp.float32)
        xws_ref[...] = xw * dis[:, None]

    return pl.pallas_call(
        body,
        out_shape=jax.ShapeDtypeStruct((n, h), jnp.float32),
    )(deg_p, x, W1)


def _tc_mid(s1_p, xws, deg_p, b1, W2p, n):
    wpad = W2p.shape[1]

    def body(s1p_ref, xws_ref, degp_ref, b1_ref, w2_ref, hws_ref):
        dp = degp_ref[...]
        deg = dp[0] + dp[1] + 1.0
        dis = lax.rsqrt(deg)[:n]
        s1 = s1p_ref[0, :n] + s1p_ref[1, :n]
        out1 = (s1 + xws_ref[...]) * dis[:, None] + b1_ref[...]
        h = jnp.maximum(out1, 0.0)
        hw = jnp.dot(h, w2_ref[...], preferred_element_type=jnp.float32)
        hws_ref[...] = hw * dis[:, None]

    return pl.pallas_call(
        body,
        out_shape=jax.ShapeDtypeStruct((n, wpad), jnp.float32),
    )(s1_p, xws, deg_p, b1, W2p)


def _tc_final(s2_p, hws, deg_p, b2, n, out_dim):
    def body(s2p_ref, hws_ref, degp_ref, b2_ref, out_ref):
        dp = degp_ref[...]
        deg = dp[0] + dp[1] + 1.0
        dis = lax.rsqrt(deg)[:n]
        s2 = s2p_ref[0, :n, :out_dim] + s2p_ref[1, :n, :out_dim]
        out_ref[...] = ((s2 + hws_ref[:, :out_dim]) * dis[:, None]
                        + b2_ref[...])

    return pl.pallas_call(
        body,
        out_shape=jax.ShapeDtypeStruct((n, out_dim), jnp.float32),
    )(s2_p, hws, deg_p, b2)


def kernel(x, edge_index, edge_weight, W1, b1, W2, b2):
    n, d = x.shape
    e = edge_index.shape[1]
    h = W1.shape[1]
    out_dim = W2.shape[1]
    w2pad = 16

    src = edge_index[0].astype(jnp.int32)
    dst = edge_index[1].astype(jnp.int32)
    ew = edge_weight.astype(jnp.float32)

    per_block = NW * CHUNK
    cpw = -(-e // per_block)
    cpw = -(-cpw // NBUF) * NBUF
    e_pad = cpw * per_block
    npad = e_pad - e
    pad_ar = jnp.arange(npad, dtype=jnp.int32)
    src_p = jnp.concatenate([src, pad_ar % n]).reshape(NW, cpw, CHUNK)
    dst_p = jnp.concatenate([dst, n + pad_ar % (N_PAD - n)]
                            ).reshape(NW, cpw, CHUNK)
    ew_p = jnp.concatenate([ew, jnp.zeros((npad,), jnp.float32)]
                           ).reshape(NW, cpw, CHUNK)

    deg_p = _make_deg_kernel(cpw)(dst_p, ew_p)
    xws = _tc_prep(deg_p, x, W1, n)
    s1_p = _make_agg_kernel(cpw, h, n)(xws, src_p, dst_p, ew_p)
    W2p = jnp.concatenate(
        [W2, jnp.zeros((h, w2pad - out_dim), jnp.float32)], axis=1)
    hws = _tc_mid(s1_p, xws, deg_p, b1.reshape(1, h), W2p, n)
    s2_p = _make_agg_kernel(cpw, w2pad, n)(hws, src_p, dst_p, ew_p)
    return _tc_final(s2_p, hws, deg_p, b2.reshape(1, out_dim), n, out_dim)
